# baseline (speedup 1.0000x reference)
import functools

import jax
import jax.numpy as jnp
from jax import lax
from jax.experimental import pallas as pl
from jax.experimental.pallas import tpu as pltpu
from jax.experimental.pallas import tpu_sc as plsc

NUM_CLASSES = 1000
WORDS_PER_ROW = NUM_CLASSES // 4
LANES = 16

NUM_CORES = 2
NUM_SUBCORES = 16
NUM_WORKERS = NUM_CORES * NUM_SUBCORES

CHUNK_ROWS = 32
N_IDX_BUFS = 4
N_OUT_BUFS = 2


def _make_multihot_kernel(batch, hist_len):
    rows_per_w = batch // NUM_WORKERS
    n_chunks = rows_per_w // CHUNK_ROWS
    n_groups = hist_len // LANES
    tail_off = hist_len - LANES
    n_wgroups = WORDS_PER_ROW // LANES
    wtail_off = WORDS_PER_ROW - LANES

    mesh = plsc.VectorSubcoreMesh(
        core_axis_name="c", subcore_axis_name="s",
        num_cores=NUM_CORES, num_subcores=NUM_SUBCORES)

    scratch = (
        [pltpu.VMEM((CHUNK_ROWS, hist_len), jnp.int32)] * N_IDX_BUFS
        + [pltpu.VMEM((CHUNK_ROWS, WORDS_PER_ROW), jnp.int32)] * N_OUT_BUFS
        + [pltpu.SemaphoreType.DMA] * (N_IDX_BUFS + N_OUT_BUFS)
    )

    @functools.partial(
        pl.kernel,
        out_type=jax.ShapeDtypeStruct((batch, WORDS_PER_ROW), jnp.int32),
        mesh=mesh,
        compiler_params=pltpu.CompilerParams(needs_layout_passes=False),
        scratch_types=scratch,
    )
    def multihot(idx_hbm, out_hbm, *scratch_refs):
        idxs = scratch_refs[:N_IDX_BUFS]
        bufs = scratch_refs[N_IDX_BUFS:N_IDX_BUFS + N_OUT_BUFS]
        sems = scratch_refs[N_IDX_BUFS + N_OUT_BUFS:]
        in_sems = sems[:N_IDX_BUFS]
        out_sems = sems[N_IDX_BUFS:]

        wid = lax.axis_index("s") * NUM_CORES + lax.axis_index("c")
        row0 = wid * rows_per_w

        zeros = jnp.zeros((LANES,), jnp.int32)
        three = jnp.full((LANES,), 3, jnp.int32)

        def in_copy(c):
            return pltpu.make_async_copy(
                idx_hbm.at[pl.ds(row0 + c * CHUNK_ROWS, CHUNK_ROWS)],
                idxs[c % N_IDX_BUFS], in_sems[c % N_IDX_BUFS])

        def out_copy(c):
            return pltpu.make_async_copy(
                bufs[c % N_OUT_BUFS],
                out_hbm.at[pl.ds(row0 + c * CHUNK_ROWS, CHUNK_ROWS)],
                out_sems[c % N_OUT_BUFS])

        def scatter_pass(buf, idxv, epoch):
            def rmw(rvec, iv):
                wv = lax.shift_right_logical(iv, 2)
                res = iv & three
                for k in range(4):
                    m = res == jnp.full((LANES,), k, jnp.int32)
                    old = plsc.load_gather(buf, [rvec, wv], mask=m)
                    new = (old & jnp.full((LANES,), ~(0xFF << (8 * k)),
                                          jnp.int32)) \
                        | jnp.full((LANES,), epoch << (8 * k), jnp.int32)
                    plsc.store_scatter(buf, [rvec, wv], new, mask=m)

            def row_body(r, _):
                rvec = zeros + r
                for g in range(n_groups):
                    rmw(rvec, idxv[r, pl.ds(g * LANES, LANES)])
                rmw(rvec, idxv[r, pl.ds(tail_off, LANES)])
                return 0
            lax.fori_loop(0, CHUNK_ROWS, row_body, 0)

        in_copy(0).start()

        def clear_body(r, _):
            for buf in bufs:
                for g in range(n_wgroups):
                    buf[r, pl.ds(g * LANES, LANES)] = zeros
                buf[r, pl.ds(wtail_off, LANES)] = zeros
            return 0
        lax.fori_loop(0, CHUNK_ROWS, clear_body, 0)

        for c in range(n_chunks):
            if c + 1 < n_chunks:
                in_copy(c + 1).start()
            if c >= N_OUT_BUFS:
                out_copy(c - N_OUT_BUFS).wait()
            in_copy(c).wait()
            scatter_pass(bufs[c % N_OUT_BUFS], idxs[c % N_IDX_BUFS], c + 1)
            out_copy(c).start()

        out_copy(n_chunks - 2).wait()
        out_copy(n_chunks - 1).wait()

    return multihot


@jax.jit
def kernel(indices):
    batch, hist_len = indices.shape
    multihot = _make_multihot_kernel(batch, hist_len)
    out = multihot(indices.astype(jnp.int32))
    by = lax.bitcast_convert_type(out, jnp.int8)
    by = by.reshape(batch, NUM_CLASSES)
    rows_per_w = batch // NUM_WORKERS
    epoch = ((jnp.arange(batch, dtype=jnp.int32) % rows_per_w)
             // CHUNK_ROWS + 1).astype(jnp.int8)
    return by == epoch[:, None]


# speedup vs baseline: 3.0938x; 3.0938x over previous
import functools

import jax
import jax.numpy as jnp
from jax import lax
from jax.experimental import pallas as pl
from jax.experimental.pallas import tpu as pltpu
from jax.experimental.pallas import tpu_sc as plsc

NUM_CLASSES = 1000
LANES = 16

NUM_CORES = 2
NUM_SUBCORES = 16
NUM_WORKERS = NUM_CORES * NUM_SUBCORES

CHUNK_ROWS = 32
N_IDX_BUFS = 4
N_OUT_BUFS = 2


def _make_multihot_kernel(batch, hist_len):
    rows_per_w = batch // NUM_WORKERS
    n_chunks = rows_per_w // CHUNK_ROWS
    n_groups = hist_len // LANES
    tail_off = hist_len - LANES
    n_cgroups = NUM_CLASSES // LANES
    ctail_off = NUM_CLASSES - LANES

    mesh = plsc.VectorSubcoreMesh(
        core_axis_name="c", subcore_axis_name="s",
        num_cores=NUM_CORES, num_subcores=NUM_SUBCORES)

    scratch = (
        [pltpu.VMEM((CHUNK_ROWS, hist_len), jnp.int32)] * N_IDX_BUFS
        + [pltpu.VMEM((CHUNK_ROWS, NUM_CLASSES), jnp.int32)] * N_OUT_BUFS
        + [pltpu.SemaphoreType.DMA] * (N_IDX_BUFS + N_OUT_BUFS)
    )

    @functools.partial(
        pl.kernel,
        out_type=jax.ShapeDtypeStruct((batch, NUM_CLASSES), jnp.int32),
        mesh=mesh,
        compiler_params=pltpu.CompilerParams(
            needs_layout_passes=False, use_tc_tiling_on_sc=True),
        scratch_types=scratch,
    )
    def multihot(idx_hbm, out_hbm, *scratch_refs):
        idxs = scratch_refs[:N_IDX_BUFS]
        bufs = scratch_refs[N_IDX_BUFS:N_IDX_BUFS + N_OUT_BUFS]
        sems = scratch_refs[N_IDX_BUFS + N_OUT_BUFS:]
        in_sems = sems[:N_IDX_BUFS]
        out_sems = sems[N_IDX_BUFS:]

        wid = lax.axis_index("s") * NUM_CORES + lax.axis_index("c")
        row0 = wid * rows_per_w

        ones = jnp.full((LANES,), 1, jnp.int32)
        zeros = jnp.zeros((LANES,), jnp.int32)

        def in_copy(c):
            return pltpu.make_async_copy(
                idx_hbm.at[pl.ds(row0 + c * CHUNK_ROWS, CHUNK_ROWS)],
                idxs[c % N_IDX_BUFS], in_sems[c % N_IDX_BUFS])

        def out_copy(c):
            return pltpu.make_async_copy(
                bufs[c % N_OUT_BUFS],
                out_hbm.at[pl.ds(row0 + c * CHUNK_ROWS, CHUNK_ROWS)],
                out_sems[c % N_OUT_BUFS])

        def scatter_pass(buf, idxv, val):
            @plsc.parallel_loop(0, CHUNK_ROWS, unroll=2)
            def row_body(r):
                rvec = zeros + r
                for g in range(n_groups):
                    iv = idxv[r, pl.ds(g * LANES, LANES)]
                    plsc.store_scatter(buf, [rvec, iv], val)
                iv = idxv[r, pl.ds(tail_off, LANES)]
                plsc.store_scatter(buf, [rvec, iv], val)

        in_copy(0).start()

        def clear_body(r, _):
            for buf in bufs:
                for g in range(n_cgroups):
                    buf[r, pl.ds(g * LANES, LANES)] = zeros
                buf[r, pl.ds(ctail_off, LANES)] = zeros
            return 0
        lax.fori_loop(0, CHUNK_ROWS, clear_body, 0)

        for c in range(n_chunks):
            if c + 1 < n_chunks:
                in_copy(c + 1).start()
            if c >= N_OUT_BUFS:
                out_copy(c - N_OUT_BUFS).wait()
            in_copy(c).wait()
            epoch = jnp.full((LANES,), c + 1, jnp.int32)
            scatter_pass(bufs[c % N_OUT_BUFS], idxs[c % N_IDX_BUFS], epoch)
            out_copy(c).start()

        out_copy(n_chunks - 2).wait()
        out_copy(n_chunks - 1).wait()

    return multihot


@jax.jit
def kernel(indices):
    batch, hist_len = indices.shape
    multihot = _make_multihot_kernel(batch, hist_len)
    out = multihot(indices.astype(jnp.int32))
    rows_per_w = batch // NUM_WORKERS
    epoch = (jnp.arange(batch, dtype=jnp.int32) % rows_per_w) // CHUNK_ROWS + 1
    return out == epoch[:, None]
